# baseline (speedup 1.0000x reference)
import functools

import jax
import jax.numpy as jnp
from jax import lax
from jax.experimental import pallas as pl
from jax.experimental.pallas import tpu as pltpu
from jax.experimental.pallas import tpu_sc as plsc

_VOCAB = 1000000
_HIDDEN = 16
_BATCH = 4096
_SEQ = 200
_B = _BATCH * _SEQ
_IW = 128
_IDX_ROWS = _B // _IW
_NC = 2
_NS = 16
_NW = _NC * _NS
_ROWS_PER_W = _IDX_ROWS // _NW
_CH = 8
_NCHUNK = _ROWS_PER_W // _CH


def _make_gather():
    mesh = plsc.VectorSubcoreMesh(core_axis_name="c", subcore_axis_name="s")

    @functools.partial(
        pl.kernel,
        mesh=mesh,
        out_type=jax.ShapeDtypeStruct((_B, _HIDDEN), jnp.float32),
        scratch_types=[
            pltpu.VMEM((_CH * _IW,), jnp.int32),
            pltpu.VMEM((_CH * _IW, _HIDDEN), jnp.float32),
            pltpu.SemaphoreType.DMA,
        ],
        compiler_params=pltpu.CompilerParams(use_tc_tiling_on_sc=False),
    )
    def gather_kernel(table_hbm, idx_hbm, out_hbm, idx_v, rows_v, sem):
        wid = lax.axis_index("s") * _NC + lax.axis_index("c")
        row_base = wid * _ROWS_PER_W

        def chunk_body(i, carry):
            roff = row_base + i * _CH
            pltpu.sync_copy(idx_hbm.at[pl.ds(roff * _IW, _CH * _IW)], idx_v)
            pltpu.async_copy(table_hbm.at[idx_v], rows_v, sem).wait()
            pltpu.sync_copy(rows_v, out_hbm.at[pl.ds(roff * _IW, _CH * _IW)])
            return carry

        lax.fori_loop(0, _NCHUNK, chunk_body, 0)

    return gather_kernel


_gather = _make_gather()


def kernel(inputs, embeddings):
    idx = inputs.reshape(_B)
    out = _gather(embeddings, idx)
    return out.reshape(_BATCH, _SEQ, _HIDDEN)


# speedup vs baseline: 1.0139x; 1.0139x over previous
import functools

import jax
import jax.numpy as jnp
from jax import lax
from jax.experimental import pallas as pl
from jax.experimental.pallas import tpu as pltpu
from jax.experimental.pallas import tpu_sc as plsc

_VOCAB = 1000000
_HIDDEN = 16
_BATCH = 4096
_SEQ = 200
_NC = 2
_NS = 16
_NW = _NC * _NS
_BPW = _BATCH // _NW
_NB = 8
_NCHUNK = _BPW // _NB


def _make_gather():
    mesh = plsc.VectorSubcoreMesh(core_axis_name="c", subcore_axis_name="s")

    @functools.partial(
        pl.kernel,
        mesh=mesh,
        out_type=jax.ShapeDtypeStruct((_BATCH, _SEQ, _HIDDEN), jnp.float32),
        scratch_types=[
            pltpu.VMEM((_NB, _SEQ), jnp.int32),
            pltpu.VMEM((_NB, _SEQ, _HIDDEN), jnp.float32),
            pltpu.SemaphoreType.DMA,
        ],
        compiler_params=pltpu.CompilerParams(use_tc_tiling_on_sc=False),
    )
    def gather_kernel(table_hbm, idx_hbm, out_hbm, idx_v, rows_v, sem):
        wid = lax.axis_index("s") * _NC + lax.axis_index("c")
        row_base = wid * _BPW

        def chunk_body(i, carry):
            boff = row_base + i * _NB
            pltpu.sync_copy(idx_hbm.at[pl.ds(boff, _NB)], idx_v)
            copies = []
            for r in range(_NB):
                copies.append(
                    pltpu.async_copy(
                        table_hbm.at[idx_v.at[r]], rows_v.at[r], sem
                    )
                )
            for c in copies:
                c.wait()
            pltpu.sync_copy(rows_v, out_hbm.at[pl.ds(boff, _NB)])
            return carry

        lax.fori_loop(0, _NCHUNK, chunk_body, 0)

    return gather_kernel


_gather = _make_gather()


def kernel(inputs, embeddings):
    return _gather(embeddings, inputs)
